# initial kernel scaffold (unmeasured)
import jax
import jax.numpy as jnp
from jax import lax
from jax.experimental import pallas as pl
from jax.experimental.pallas import tpu as pltpu


def kernel(
    x,
):
    def body(*refs):
        pass

    out_shape = jax.ShapeDtypeStruct(..., jnp.float32)
    return pl.pallas_call(body, out_shape=out_shape)(...)


# baseline (device time: 428476 ns/iter reference)
import jax
import jax.numpy as jnp
from jax import lax
from jax.experimental import pallas as pl
from jax.experimental.pallas import tpu as pltpu

N_DEV = 32


def _ring_allgather(x_shard):
    m_per, n = x_shard.shape

    def body(x_ref, out_ref, send_sems, recv_sems):
        my_pos = lax.axis_index("i")
        left = lax.rem(my_pos - 1 + N_DEV, N_DEV)
        right = lax.rem(my_pos + 1, N_DEV)

        barrier_sem = pltpu.get_barrier_semaphore()
        for nbr in (left, right):
            pl.semaphore_signal(
                barrier_sem, inc=1,
                device_id=(nbr,), device_id_type=pl.DeviceIdType.MESH,
            )
        pl.semaphore_wait(barrier_sem, 2)

        out_ref[pl.ds(my_pos * m_per, m_per), :] = x_ref[:, :]

        for h in range(N_DEV - 1):
            src_origin = lax.rem(my_pos - h + N_DEV, N_DEV)
            rdma = pltpu.make_async_remote_copy(
                src_ref=out_ref.at[pl.ds(src_origin * m_per, m_per), :],
                dst_ref=out_ref.at[pl.ds(src_origin * m_per, m_per), :],
                send_sem=send_sems.at[h],
                recv_sem=recv_sems.at[h],
                device_id=(right,),
                device_id_type=pl.DeviceIdType.MESH,
            )
            rdma.start()
            rdma.wait()

    return pl.pallas_call(
        body,
        out_shape=jax.ShapeDtypeStruct((N_DEV * m_per, n), x_shard.dtype),
        in_specs=[pl.BlockSpec(memory_space=pltpu.VMEM)],
        out_specs=pl.BlockSpec(memory_space=pltpu.VMEM),
        scratch_shapes=[
            pltpu.SemaphoreType.DMA((N_DEV - 1,)),
            pltpu.SemaphoreType.DMA((N_DEV - 1,)),
        ],
        compiler_params=pltpu.CompilerParams(collective_id=0),
    )(x_shard)


def kernel(x):
    m_per, n = x.shape
    xb = x.astype(jnp.bfloat16)
    gathered = _ring_allgather(xb)
    s = jnp.sort(gathered, axis=0)
    my_pos = lax.axis_index("i")
    out = lax.dynamic_slice_in_dim(s, my_pos * m_per, m_per, axis=0)
    return out.astype(jnp.float32)


# device time: 50670 ns/iter; 8.4562x vs baseline; 8.4562x over previous
import jax
import jax.numpy as jnp
from jax import lax
from jax.experimental import pallas as pl
from jax.experimental.pallas import tpu as pltpu

N_DEV = 32
M_PER = 256
N_ROUNDS = 15


def _cmpex(v, d, asc):
    r, n = v.shape
    b = r // (2 * d)
    v4 = v.reshape(b, 2, d, n)
    a, c = v4[:, 0], v4[:, 1]
    mn, mx = jnp.minimum(a, c), jnp.maximum(a, c)
    lo = jnp.where(asc, mn, mx)
    hi = jnp.where(asc, mx, mn)
    return jnp.stack([lo, hi], axis=1).reshape(r, n)


def _static_asc(d, m, r):
    b = r // (2 * d)
    g = lax.broadcasted_iota(jnp.int32, (b, 1, 1), 0)
    return ((g * (2 * d)) // m) % 2 == 0


def kernel(x):
    m_per, n = x.shape
    assert m_per == M_PER

    def body(x_ref, out_ref, blk_ref, recv_ref, send_sems, recv_sems):
        p = lax.axis_index("i")

        barrier_sem = pltpu.get_barrier_semaphore()
        for c in (1, 2, 4, 8, 16):
            pl.semaphore_signal(
                barrier_sem, inc=1,
                device_id=(jnp.bitwise_xor(p, c),),
                device_id_type=pl.DeviceIdType.MESH,
            )
        pl.semaphore_wait(barrier_sem, 5)

        v = x_ref[:, :].astype(jnp.bfloat16).astype(jnp.float32)
        m = 2
        while m <= m_per // 2:
            d = m // 2
            while d >= 1:
                v = _cmpex(v, d, _static_asc(d, m, m_per))
                d //= 2
            m *= 2
        asc_p = (p % 2) == 0
        d = m_per // 2
        while d >= 1:
            v = _cmpex(v, d, asc_p)
            d //= 2
        blk_ref[:, :] = v.astype(jnp.bfloat16)

        rnd = 0
        m = 2 * m_per
        while m <= N_DEV * m_per:
            chips_per_block = m // m_per
            asc_p = (p // chips_per_block) % 2 == 0
            c = m // (2 * m_per)
            while c >= 1:
                partner = jnp.bitwise_xor(p, c)
                rdma = pltpu.make_async_remote_copy(
                    src_ref=blk_ref,
                    dst_ref=recv_ref.at[rnd],
                    send_sem=send_sems.at[rnd],
                    recv_sem=recv_sems.at[rnd],
                    device_id=(partner,),
                    device_id_type=pl.DeviceIdType.MESH,
                )
                rdma.start()
                rdma.wait()
                mine = blk_ref[:, :]
                theirs = recv_ref[rnd, :, :]
                take_min = ((p & c) == 0) == asc_p
                blk_ref[:, :] = jnp.where(
                    take_min,
                    jnp.minimum(mine, theirs),
                    jnp.maximum(mine, theirs),
                )
                rnd += 1
                c //= 2
            v = blk_ref[:, :].astype(jnp.float32)
            d = m_per // 2
            while d >= 1:
                v = _cmpex(v, d, asc_p)
                d //= 2
            blk_ref[:, :] = v.astype(jnp.bfloat16)
            m *= 2

        out_ref[:, :] = blk_ref[:, :].astype(jnp.float32)

    return pl.pallas_call(
        body,
        out_shape=jax.ShapeDtypeStruct((m_per, n), jnp.float32),
        in_specs=[pl.BlockSpec(memory_space=pltpu.VMEM)],
        out_specs=pl.BlockSpec(memory_space=pltpu.VMEM),
        scratch_shapes=[
            pltpu.VMEM((m_per, n), jnp.bfloat16),
            pltpu.VMEM((N_ROUNDS, m_per, n), jnp.bfloat16),
            pltpu.SemaphoreType.DMA((N_ROUNDS,)),
            pltpu.SemaphoreType.DMA((N_ROUNDS,)),
        ],
        compiler_params=pltpu.CompilerParams(collective_id=0),
    )(x)


# device time: 46184 ns/iter; 9.2776x vs baseline; 1.0971x over previous
import jax
import jax.numpy as jnp
from jax import lax
from jax.experimental import pallas as pl
from jax.experimental.pallas import tpu as pltpu

N_DEV = 32
M_PER = 256


def _round_schedule():
    rounds = []
    k = 2
    while k <= N_DEV:
        dists = []
        c = k // 2
        while c >= 1:
            dists.append(c)
            c //= 2
        i = 0
        while i < len(dists):
            if i + 1 < len(dists):
                c2, c1 = dists[i], dists[i + 1]
                rounds.append((k, [c1, c2, c1 + c2]))
                i += 2
            else:
                rounds.append((k, [dists[i]]))
                i += 1
        k *= 2
    return rounds


ROUNDS = _round_schedule()
N_SLOTS = sum(len(masks) for _, masks in ROUNDS)
BARRIER_MASKS = sorted({m for _, masks in ROUNDS for m in masks})


def _cmpex(v, d, asc):
    r, n = v.shape
    b = r // (2 * d)
    v4 = v.reshape(b, 2, d, n)
    a, c = v4[:, 0], v4[:, 1]
    mn, mx = jnp.minimum(a, c), jnp.maximum(a, c)
    lo = jnp.where(asc, mn, mx)
    hi = jnp.where(asc, mx, mn)
    return jnp.stack([lo, hi], axis=1).reshape(r, n)


def _static_asc(d, m, r):
    b = r // (2 * d)
    g = lax.broadcasted_iota(jnp.int32, (b, 1, 1), 0)
    return ((g * (2 * d)) // m) % 2 == 0


def kernel(x):
    m_per, n = x.shape
    assert m_per == M_PER

    def body(x_ref, out_ref, blk_ref, recv_ref, send_sems, recv_sems):
        p = lax.axis_index("i")

        barrier_sem = pltpu.get_barrier_semaphore()
        for c in BARRIER_MASKS:
            pl.semaphore_signal(
                barrier_sem, inc=1,
                device_id=(jnp.bitwise_xor(p, c),),
                device_id_type=pl.DeviceIdType.MESH,
            )
        pl.semaphore_wait(barrier_sem, len(BARRIER_MASKS))

        v = x_ref[:, :].astype(jnp.bfloat16).astype(jnp.float32)
        m = 2
        while m <= m_per // 2:
            d = m // 2
            while d >= 1:
                v = _cmpex(v, d, _static_asc(d, m, m_per))
                d //= 2
            m *= 2
        asc_p = (p % 2) == 0
        d = m_per // 2
        while d >= 1:
            v = _cmpex(v, d, asc_p)
            d //= 2
        blk_ref[:, :] = v.astype(jnp.bfloat16)

        slot = 0
        prev_k = None
        for idx, (k, masks) in enumerate(ROUNDS):
            asc_p = (p // k) % 2 == 0
            rdmas = []
            for j, mk in enumerate(masks):
                rdma = pltpu.make_async_remote_copy(
                    src_ref=blk_ref,
                    dst_ref=recv_ref.at[slot + j],
                    send_sem=send_sems.at[slot + j],
                    recv_sem=recv_sems.at[slot + j],
                    device_id=(jnp.bitwise_xor(p, mk),),
                    device_id_type=pl.DeviceIdType.MESH,
                )
                rdma.start()
                rdmas.append(rdma)
            for rdma in rdmas:
                rdma.wait()
            mine = blk_ref[:, :]
            if len(masks) == 3:
                c1, c2 = masks[0], masks[1]
                t1 = recv_ref[slot + 0, :, :]
                t2 = recv_ref[slot + 1, :, :]
                t3 = recv_ref[slot + 2, :, :]
                sel1 = ((p & c2) == 0) == asc_p
                a = jnp.where(sel1, jnp.minimum(mine, t2),
                              jnp.maximum(mine, t2))
                b = jnp.where(sel1, jnp.minimum(t1, t3),
                              jnp.maximum(t1, t3))
                sel2 = ((p & c1) == 0) == asc_p
                blk_ref[:, :] = jnp.where(sel2, jnp.minimum(a, b),
                                          jnp.maximum(a, b))
            else:
                c = masks[0]
                theirs = recv_ref[slot, :, :]
                take_min = ((p & c) == 0) == asc_p
                blk_ref[:, :] = jnp.where(
                    take_min,
                    jnp.minimum(mine, theirs),
                    jnp.maximum(mine, theirs),
                )
            slot += len(masks)
            if idx + 1 == len(ROUNDS) or ROUNDS[idx + 1][0] != k:
                v = blk_ref[:, :].astype(jnp.float32)
                d = m_per // 2
                while d >= 1:
                    v = _cmpex(v, d, asc_p)
                    d //= 2
                blk_ref[:, :] = v.astype(jnp.bfloat16)

        out_ref[:, :] = blk_ref[:, :].astype(jnp.float32)

    return pl.pallas_call(
        body,
        out_shape=jax.ShapeDtypeStruct((m_per, n), jnp.float32),
        in_specs=[pl.BlockSpec(memory_space=pltpu.VMEM)],
        out_specs=pl.BlockSpec(memory_space=pltpu.VMEM),
        scratch_shapes=[
            pltpu.VMEM((m_per, n), jnp.bfloat16),
            pltpu.VMEM((N_SLOTS, m_per, n), jnp.bfloat16),
            pltpu.SemaphoreType.DMA((N_SLOTS,)),
            pltpu.SemaphoreType.DMA((N_SLOTS,)),
        ],
        compiler_params=pltpu.CompilerParams(collective_id=0),
    )(x)
